# baseline (speedup 1.0000x reference)
import functools

import jax
import jax.numpy as jnp
from jax import lax
from jax.experimental import pallas as pl
from jax.experimental.pallas import tpu as pltpu
from jax.experimental.pallas import tpu_sc as plsc


def _make_sc_kernel(Lx, Nx, M):
    D = Lx - 2
    LANES = 16
    PER_W = D * LANES
    CH = 80
    NCH = PER_W // CH
    assert PER_W % CH == 0 and CH % 8 == 0

    mesh = plsc.VectorSubcoreMesh(core_axis_name="c", subcore_axis_name="s")

    @functools.partial(
        pl.kernel,
        mesh=mesh,
        out_type=[
            jax.ShapeDtypeStruct((LANES, LANES), jnp.float32),
            jax.ShapeDtypeStruct((LANES, LANES), jnp.float32),
        ],
        scratch_types=[
            pltpu.VMEM((D, LANES), jnp.int32),
            pltpu.VMEM((PER_W,), jnp.int32),
            pltpu.VMEM((PER_W,), jnp.float32),
            pltpu.VMEM((PER_W,), jnp.float32),
            pltpu.VMEM((LANES,), jnp.float32),
            pltpu.VMEM((LANES,), jnp.float32),
            pltpu.SemaphoreType.DMA,
        ],
    )
    def crit_kernel(inp_hbm, seqr_hbm, osum_hbm, ocnt_hbm, seq_v, idx_v,
                    msk_v, val_v, acc_v, cnt_v, sem):
        c = lax.axis_index("c")
        s = lax.axis_index("s")

        @pl.when(c == 0)
        def _core0():
            pltpu.sync_copy(seqr_hbm.at[s], seq_v)
            lanes = lax.iota(jnp.int32, LANES)
            colbase = (s * LANES + lanes) * M
            seen = lanes * 0
            for t in range(D):
                sv = seq_v[t, :]
                nz = jnp.minimum(sv, 1)
                first0 = (1 - nz) * (1 - seen)
                seen = jnp.maximum(seen, 1 - nz)
                eff = sv + first0 * (M - 1)
                off = (t + 1) * Nx * M + colbase + eff
                mskf = jnp.minimum(eff, 1).astype(jnp.float32)
                idx_v[pl.ds(t * LANES, LANES)] = off
                msk_v[pl.ds(t * LANES, LANES)] = mskf

            copies = [
                pltpu.async_copy(
                    inp_hbm.at[idx_v.at[pl.ds(k * CH, CH)]],
                    val_v.at[pl.ds(k * CH, CH)],
                    sem,
                )
                for k in range(NCH)
            ]
            for cp in copies:
                cp.wait()

            acc = None
            cnt = None
            for t in range(D):
                vals = val_v[pl.ds(t * LANES, LANES)]
                m = msk_v[pl.ds(t * LANES, LANES)]
                term = vals * m
                acc = term if acc is None else acc + term
                cnt = m if cnt is None else cnt + m
            acc_v[...] = acc
            cnt_v[...] = cnt
            pltpu.sync_copy(acc_v, osum_hbm.at[s])
            pltpu.sync_copy(cnt_v, ocnt_hbm.at[s])

    return crit_kernel


def _fold_kernel(sum_ref, cnt_ref, out_ref):
    total = jnp.sum(sum_ref[...])
    count = jnp.sum(cnt_ref[...])
    out_ref[...] = jnp.full((1, 1), -(total / count), jnp.float32)


def kernel(input, seq):
    Lx, Nx, M = input.shape
    D = Lx - 2
    inp_flat = input.reshape(-1)
    seqr = seq.reshape(D, Nx // 16, 16).transpose(1, 0, 2)
    psum, pcnt = _make_sc_kernel(Lx, Nx, M)(inp_flat, seqr)
    loss = pl.pallas_call(
        _fold_kernel,
        out_shape=jax.ShapeDtypeStruct((1, 1), jnp.float32),
    )(psum, pcnt)
    return loss[0, 0]


# speedup vs baseline: 1.4502x; 1.4502x over previous
import functools

import jax
import jax.numpy as jnp
from jax import lax
from jax.experimental import pallas as pl
from jax.experimental.pallas import tpu as pltpu


def _make_tc_kernel(Lx, Nx, M):
    D = Lx - 2

    def body(seq_ref, x_ref, out_ref, acc_ref):
        t = pl.program_id(0)

        @pl.when(t == 0)
        def _init():
            acc_ref[0] = jnp.float32(0)
            acc_ref[1] = jnp.float32(0)

        seq = seq_ref[...]
        nz = jnp.minimum(seq, 1)
        r = lax.broadcasted_iota(jnp.int32, (D, D), 0)
        c = lax.broadcasted_iota(jnp.int32, (D, D), 1)
        tri = (r >= c).astype(jnp.float32)
        zcount = jnp.dot(tri, (1 - nz).astype(jnp.float32),
                         preferred_element_type=jnp.float32)
        first0 = (1 - nz) * jnp.where(zcount == 1.0, 1, 0)
        eff = seq + first0 * (M - 1)
        rowsel = lax.broadcasted_iota(jnp.int32, (D, Nx), 0) == t
        eff_t = jnp.sum(jnp.where(rowsel, eff, 0), axis=0,
                        keepdims=True)
        msk_t = jnp.minimum(eff_t, 1).astype(jnp.float32)

        x = x_ref[0]
        sel = lax.broadcasted_iota(jnp.int32, (Nx, M), 1) == eff_t.reshape(Nx, 1)
        vals = jnp.sum(jnp.where(sel, x, jnp.float32(0)), axis=1)
        acc_ref[0] += jnp.sum(vals * msk_t[0])
        acc_ref[1] += jnp.sum(msk_t)

        @pl.when(t == D - 1)
        def _fin():
            out_ref[...] = jnp.full((1, 1), -(acc_ref[0] / acc_ref[1]),
                                    jnp.float32)

    return pl.pallas_call(
        body,
        grid=(D,),
        in_specs=[
            pl.BlockSpec((D, Nx), lambda t: (0, 0)),
            pl.BlockSpec((1, Nx, M), lambda t: (t + 1, 0, 0)),
        ],
        out_specs=pl.BlockSpec((1, 1), lambda t: (0, 0)),
        out_shape=jax.ShapeDtypeStruct((1, 1), jnp.float32),
        scratch_shapes=[pltpu.SMEM((2,), jnp.float32)],
    )


def kernel(input, seq):
    Lx, Nx, M = input.shape
    out = _make_tc_kernel(Lx, Nx, M)(seq, input)
    return out[0, 0]


# speedup vs baseline: 1.4547x; 1.0031x over previous
import functools

import jax
import jax.numpy as jnp
from jax import lax
from jax.experimental import pallas as pl
from jax.experimental.pallas import tpu as pltpu

_NSTREAM = 4


def _make_tc_kernel(Lx, Nx, M):
    D = Lx - 2
    NB = Nx // _NSTREAM

    def body(seq_ref, *rest):
        x_refs = rest[:_NSTREAM]
        out_ref = rest[_NSTREAM]
        acc_ref = rest[_NSTREAM + 1]
        t = pl.program_id(0)

        @pl.when(t == 0)
        def _init():
            acc_ref[0] = jnp.float32(0)
            acc_ref[1] = jnp.float32(0)

        seq = seq_ref[...]
        nz = jnp.minimum(seq, 1)
        r = lax.broadcasted_iota(jnp.int32, (D, D), 0)
        c = lax.broadcasted_iota(jnp.int32, (D, D), 1)
        tri = (r >= c).astype(jnp.float32)
        zcount = jnp.dot(tri, (1 - nz).astype(jnp.float32),
                         preferred_element_type=jnp.float32)
        first0 = (1 - nz) * jnp.where(zcount == 1.0, 1, 0)
        eff = seq + first0 * (M - 1)
        rowsel = lax.broadcasted_iota(jnp.int32, (D, Nx), 0) == t
        eff_t = jnp.sum(jnp.where(rowsel, eff, 0), axis=0)
        msk_t = jnp.minimum(eff_t, 1).astype(jnp.float32)

        lane = lax.broadcasted_iota(jnp.int32, (NB, M), 1)
        total = acc_ref[0]
        count = acc_ref[1]
        for k in range(_NSTREAM):
            x = x_refs[k][0]
            eff_k = eff_t[k * NB:(k + 1) * NB].reshape(NB, 1)
            msk_k = msk_t[k * NB:(k + 1) * NB]
            sel = lane == eff_k
            vals = jnp.sum(jnp.where(sel, x, jnp.float32(0)), axis=1)
            total = total + jnp.sum(vals * msk_k)
            count = count + jnp.sum(msk_k)
        acc_ref[0] = total
        acc_ref[1] = count

        @pl.when(t == D - 1)
        def _fin():
            out_ref[...] = jnp.full((1, 1), -(acc_ref[0] / acc_ref[1]),
                                    jnp.float32)

    in_specs = [pl.BlockSpec((D, Nx), lambda t: (0, 0))]
    for k in range(_NSTREAM):
        in_specs.append(
            pl.BlockSpec((1, NB, M), functools.partial(
                lambda t, kk: (t + 1, kk, 0), kk=k)))

    return pl.pallas_call(
        body,
        grid=(D,),
        in_specs=in_specs,
        out_specs=pl.BlockSpec((1, 1), lambda t: (0, 0)),
        out_shape=jax.ShapeDtypeStruct((1, 1), jnp.float32),
        scratch_shapes=[pltpu.SMEM((2,), jnp.float32)],
    )


def kernel(input, seq):
    Lx, Nx, M = input.shape
    xs = (input,) * _NSTREAM
    out = _make_tc_kernel(Lx, Nx, M)(seq, *xs)
    return out[0, 0]
